# Initial kernel scaffold; baseline (speedup 1.0000x reference)
import jax
import jax.numpy as jnp
from jax.experimental import pallas as pl


def kernel(x, table):
    raise NotImplementedError("write your pallas kernel here")


# speedup vs baseline: 1.8391x; 1.8391x over previous
import functools

import jax
import jax.numpy as jnp
from jax import lax
from jax.experimental import pallas as pl
from jax.experimental.pallas import tpu as pltpu
from jax.experimental.pallas import tpu_sc as plsc

NC = 2
NS = 16
NW = NC * NS
CH = 128


def _gather_body(n_chunks, table_hbm, idx_hbm, out_hbm, idx_v, rows_v, gsems):
    wid = lax.axis_index("s") * NC + lax.axis_index("c")
    base = wid * (n_chunks * CH)

    pltpu.sync_copy(idx_hbm.at[wid], idx_v)

    pltpu.async_copy(table_hbm.at[idx_v.at[0]], rows_v.at[0], gsems.at[0])

    def step(j, carry):
        buf = lax.rem(j, 2)
        nbuf = lax.rem(j + 1, 2)

        @pl.when(j + 1 < n_chunks)
        def _():
            pltpu.async_copy(
                table_hbm.at[idx_v.at[j + 1]], rows_v.at[nbuf], gsems.at[nbuf]
            )

        pltpu.make_async_copy(
            table_hbm.at[idx_v.at[j]], rows_v.at[buf], gsems.at[buf]
        ).wait()
        pltpu.sync_copy(rows_v.at[buf], out_hbm.at[pl.ds(base + j * CH, CH)])
        return carry

    lax.fori_loop(0, n_chunks, step, 0)


def kernel(x, table):
    b, l = x.shape
    emb = table.shape[1]
    total = b * l
    assert total % (NW * CH) == 0
    n_chunks = total // (NW * CH)

    idx = x.reshape(NW, n_chunks, CH).astype(jnp.int32)

    mesh = plsc.VectorSubcoreMesh(
        core_axis_name="c", subcore_axis_name="s", num_cores=NC, num_subcores=NS
    )
    run = pl.kernel(
        functools.partial(_gather_body, n_chunks),
        out_type=jax.ShapeDtypeStruct((total, emb), table.dtype),
        mesh=mesh,
        scratch_types=[
            pltpu.VMEM((n_chunks, CH), jnp.int32),
            pltpu.VMEM((2, CH, emb), jnp.float32),
            pltpu.SemaphoreType.DMA((2,)),
        ],
        compiler_params=pltpu.CompilerParams(use_tc_tiling_on_sc=False),
    )
    out = run(table, idx)
    return out.reshape(b, l, emb)
